# baseline (speedup 1.0000x reference)
import functools
import math

import jax
import jax.numpy as jnp
from jax import lax
from jax.experimental import pallas as pl
from jax.experimental.pallas import tpu as pltpu
from jax.experimental.pallas import tpu_sc as plsc

N = 10000
H = 128
G = 64
L = 4
EPS = 1e-5
CBN = 1.0 / math.sqrt(1.0 + EPS)
NEG = -3.0e38

NC, NS = 2, 16
NW = NC * NS
NP = 10240
RPT = NP // NS
CH = 128
NCHUNK = 80
EPT = NCHUNK * CH
EPAD = NW * EPT
HB = H // 2

BLK = 512
NBLK = NP // BLK

_mesh = plsc.VectorSubcoreMesh(core_axis_name="c", subcore_axis_name="s")


def _zero_vmem_rows(ref, nrows, width):
    zz = jnp.zeros((16,), jnp.float32)

    def body(r, _):
        for k in range(width // 16):
            ref[r, pl.ds(k * 16, 16)] = zz
        return 0

    lax.fori_loop(0, nrows, body, 0)


@functools.partial(
    pl.kernel,
    out_type=jax.ShapeDtypeStruct((NC, NP), jnp.float32),
    mesh=_mesh,
    scratch_types=[
        pltpu.VMEM((NCHUNK, CH), jnp.int32),
        pltpu.VMEM((CH,), jnp.float32),
        pltpu.VMEM((RPT,), jnp.float32),
        pltpu.VMEM_SHARED((NP,), jnp.float32),
    ],
    compiler_params=pltpu.CompilerParams(use_tc_tiling_on_sc=False),
)
def _deg_kernel(dst_hbm, out_hbm, idx_v, ones_v, zrow_v, acc_sh):
    c = lax.axis_index("c")
    s = lax.axis_index("s")
    pltpu.sync_copy(dst_hbm.at[c, s], idx_v)
    one = jnp.ones((16,), jnp.float32)
    zero = jnp.zeros((16,), jnp.float32)
    for k in range(CH // 16):
        ones_v[pl.ds(k * 16, 16)] = one

    def zb(i, _):
        zrow_v[pl.ds(i * 16, 16)] = zero
        return 0

    lax.fori_loop(0, RPT // 16, zb, 0)
    pltpu.sync_copy(zrow_v, acc_sh.at[pl.ds(s * RPT, RPT)])
    plsc.subcore_barrier()

    def body(j, _):
        pltpu.sync_copy(ones_v, acc_sh.at[idx_v.at[j]], add=True)
        return 0

    lax.fori_loop(0, NCHUNK, body, 0)
    plsc.subcore_barrier()
    pltpu.sync_copy(acc_sh.at[pl.ds(s * RPT, RPT)], out_hbm.at[c, pl.ds(s * RPT, RPT)])


@functools.partial(
    pl.kernel,
    out_type=jax.ShapeDtypeStruct((2, NC, NP, HB), jnp.float32),
    mesh=_mesh,
    scratch_types=[
        pltpu.VMEM((NCHUNK, CH), jnp.int32),
        pltpu.VMEM((NCHUNK, CH), jnp.int32),
        pltpu.VMEM((CH, HB), jnp.float32),
        pltpu.VMEM((CH, HB), jnp.float32),
        pltpu.VMEM((CH, HB), jnp.float32),
        pltpu.VMEM_SHARED((NP, HB), jnp.float32),
        pltpu.SemaphoreType.DMA,
        pltpu.SemaphoreType.DMA,
    ],
    compiler_params=pltpu.CompilerParams(use_tc_tiling_on_sc=False),
)
def _edge_kernel(u_hbm, src0_hbm, src1_hbm, dst_hbm, out_hbm, src_v, dst_v,
                 rows0_v, rows1_v, zbuf_v, acc_sh, sem0, sem1):
    c = lax.axis_index("c")
    s = lax.axis_index("s")
    pltpu.sync_copy(dst_hbm.at[c, s], dst_v)
    _zero_vmem_rows(zbuf_v, CH, HB)

    for p, src_hbm in enumerate((src0_hbm, src1_hbm)):
        pltpu.sync_copy(src_hbm.at[c, s], src_v)
        for j in range(RPT // CH):
            pltpu.sync_copy(zbuf_v, acc_sh.at[pl.ds(s * RPT + j * CH, CH)])
        plsc.subcore_barrier()

        pltpu.async_copy(u_hbm.at[src_v.at[0]], rows0_v, sem0)
        pltpu.async_copy(u_hbm.at[src_v.at[1]], rows1_v, sem1)

        def body(jj, _):
            j = 2 * jj
            pltpu.make_async_copy(u_hbm.at[src_v.at[j]], rows0_v, sem0).wait()
            pltpu.sync_copy(rows0_v, acc_sh.at[dst_v.at[j]], add=True)
            pltpu.async_copy(u_hbm.at[src_v.at[j + 2]], rows0_v, sem0)
            pltpu.make_async_copy(u_hbm.at[src_v.at[j]], rows1_v, sem1).wait()
            pltpu.sync_copy(rows1_v, acc_sh.at[dst_v.at[j + 1]], add=True)
            pltpu.async_copy(u_hbm.at[src_v.at[j + 3]], rows1_v, sem1)
            return 0

        lax.fori_loop(0, NCHUNK // 2 - 1, body, 0)
        pltpu.make_async_copy(u_hbm.at[src_v.at[0]], rows0_v, sem0).wait()
        pltpu.sync_copy(rows0_v, acc_sh.at[dst_v.at[NCHUNK - 2]], add=True)
        pltpu.make_async_copy(u_hbm.at[src_v.at[0]], rows1_v, sem1).wait()
        pltpu.sync_copy(rows1_v, acc_sh.at[dst_v.at[NCHUNK - 1]], add=True)

        plsc.subcore_barrier()
        for j in range(RPT // CH):
            r0 = s * RPT + j * CH
            pltpu.sync_copy(acc_sh.at[pl.ds(r0, CH)], out_hbm.at[p, c, pl.ds(r0, CH)])


@functools.partial(
    pl.kernel,
    out_type=(
        jax.ShapeDtypeStruct((G, H), jnp.float32),
        jax.ShapeDtypeStruct((G, H), jnp.float32),
    ),
    mesh=_mesh,
    scratch_types=[
        pltpu.VMEM((NP,), jnp.int32),
        pltpu.VMEM((16, H), jnp.float32),
        pltpu.VMEM((2, H), jnp.float32),
        pltpu.VMEM((2, H), jnp.float32),
    ],
    compiler_params=pltpu.CompilerParams(
        use_tc_tiling_on_sc=False, needs_layout_passes=False),
)
def _pool_kernel(h_hbm, batch_hbm, mean_hbm, mx_hbm, batch_v, rowbuf, mbuf, xbuf):
    c = lax.axis_index("c")
    s = lax.axis_index("s")
    wid = s * NC + c
    g0 = 2 * wid
    pltpu.sync_copy(batch_hbm, batch_v)
    i0 = jnp.int32(0)
    i1 = jnp.int32(1)

    def cnt_body(i, carry):
        lt0, n0, n1 = carry
        b = batch_v[pl.ds(i * 16, 16)]
        vg0 = jnp.full((16,), g0, jnp.int32)
        vg1 = jnp.full((16,), g0 + 1, jnp.int32)
        vz = jnp.zeros((16,), jnp.int32)
        vo = jnp.full((16,), 1, jnp.int32)
        lt = jnp.minimum(jnp.maximum(vg0 - b, vz), vo)
        e0 = vo - jnp.minimum(jnp.abs(b - vg0), vo)
        e1 = vo - jnp.minimum(jnp.abs(b - vg1), vo)
        return lt0 + jnp.sum(lt), n0 + jnp.sum(e0), n1 + jnp.sum(e1)

    lt0, n0, n1 = lax.fori_loop(0, NP // 16, cnt_body, (i0, i0, i0))

    def seg_reduce(start, n, row):
        zero = jnp.zeros((16,), jnp.float32)
        negv = jnp.full((16,), NEG, jnp.float32)
        for k in range(H // 16):
            mbuf[row, pl.ds(k * 16, 16)] = zero
            xbuf[row, pl.ds(k * 16, 16)] = negv

        def ch_body(j, _):
            pltpu.sync_copy(h_hbm.at[pl.ds(start + j * 16, 16)], rowbuf)
            vz = jnp.zeros((16,), jnp.int32)
            vo = jnp.full((16,), 1, jnp.int32)
            vn = jnp.full((16,), n, jnp.int32)
            for r in range(16):
                vr = jnp.full((16,), j * 16 + r, jnp.int32)
                vf = jnp.minimum(jnp.maximum(vn - vr, vz), vo).astype(jnp.float32)
                off = (1.0 - vf) * NEG
                for k in range(H // 16):
                    v = rowbuf[r, pl.ds(k * 16, 16)]
                    mbuf[row, pl.ds(k * 16, 16)] = mbuf[row, pl.ds(k * 16, 16)] + v * vf
                    xbuf[row, pl.ds(k * 16, 16)] = jnp.maximum(
                        xbuf[row, pl.ds(k * 16, 16)], v * vf + off)
            return 0

        nch = lax.shift_right_logical(n + 15, 4)
        lax.fori_loop(0, nch, ch_body, 0)
        vn = jnp.full((16,), n, jnp.int32)
        nzv = jnp.minimum(vn, jnp.full((16,), 1, jnp.int32)).astype(jnp.float32)
        inv = nzv / jnp.maximum(vn.astype(jnp.float32), jnp.full((16,), 1.0))
        for k in range(H // 16):
            mbuf[row, pl.ds(k * 16, 16)] = mbuf[row, pl.ds(k * 16, 16)] * inv
            xbuf[row, pl.ds(k * 16, 16)] = xbuf[row, pl.ds(k * 16, 16)] * nzv

    seg_reduce(lt0, n0, 0)
    seg_reduce(lt0 + n0, n1, 1)
    pltpu.sync_copy(mbuf, mean_hbm.at[pl.ds(g0, 2)])
    pltpu.sync_copy(xbuf, mx_hbm.at[pl.ds(g0, 2)])


def _row_spec(width=H):
    return pl.BlockSpec((BLK, width), lambda i: (i, 0))


def _full_spec(shape):
    nd = len(shape)
    return pl.BlockSpec(shape, lambda i: (0,) * nd)


def _dinv(d0, d1):
    return lax.rsqrt(d0 + d1 + 1.0)


def _in_body(x_ref, win_ref, bin_ref, w1_ref, d0_ref, d1_ref, h_ref, u_ref):
    h = jnp.maximum(jnp.dot(x_ref[...], win_ref[...],
                            preferred_element_type=jnp.float32) + bin_ref[...], 0.0)
    dinv = _dinv(d0_ref[...], d1_ref[...])
    h_ref[...] = h
    u_ref[...] = dinv * jnp.dot(h, w1_ref[...], preferred_element_type=jnp.float32)


def _acc_full(a00_ref, a01_ref, a10_ref, a11_ref):
    return jnp.concatenate(
        [a00_ref[...] + a01_ref[...], a10_ref[...] + a11_ref[...]], axis=1)


def _fuse_body(a00_ref, a01_ref, a10_ref, a11_ref, u_ref, res_ref, d0_ref,
               d1_ref, b_ref, g_ref, bt_ref, wn_ref, h_ref, un_ref):
    dinv = _dinv(d0_ref[...], d1_ref[...])
    acc = _acc_full(a00_ref, a01_ref, a10_ref, a11_ref)
    t = dinv * (acc + u_ref[...]) + b_ref[...]
    t = t * (CBN * g_ref[...]) + bt_ref[...]
    h = jnp.maximum(t, 0.0) + res_ref[...]
    h_ref[...] = h
    un_ref[...] = dinv * jnp.dot(h, wn_ref[...], preferred_element_type=jnp.float32)


def _last_body(a00_ref, a01_ref, a10_ref, a11_ref, u_ref, res_ref, d0_ref,
               d1_ref, b_ref, g_ref, bt_ref, h_ref):
    dinv = _dinv(d0_ref[...], d1_ref[...])
    acc = _acc_full(a00_ref, a01_ref, a10_ref, a11_ref)
    t = dinv * (acc + u_ref[...]) + b_ref[...]
    t = t * (CBN * g_ref[...]) + bt_ref[...]
    h_ref[...] = jnp.maximum(t, 0.0) + res_ref[...]


def _mlp_body(mean_ref, mx_ref, w1a_ref, w1b_ref, b1_ref, g1_ref, bt1_ref,
              w2_ref, b2_ref, g2_ref, bt2_ref, w3_ref, b3_ref, out_ref):
    z = (jnp.dot(mean_ref[...], w1a_ref[...], preferred_element_type=jnp.float32)
         + jnp.dot(mx_ref[...], w1b_ref[...], preferred_element_type=jnp.float32)
         + b1_ref[...])
    z = jnp.maximum(z * (CBN * g1_ref[...]) + bt1_ref[...], 0.0)
    z = jnp.dot(z, w2_ref[...], preferred_element_type=jnp.float32) + b2_ref[...]
    z = jnp.maximum(z * (CBN * g2_ref[...]) + bt2_ref[...], 0.0)
    out_ref[...] = jnp.dot(z, w3_ref[...], preferred_element_type=jnp.float32) + b3_ref[...]


def _tc_call(body, in_arrays, in_specs, out_specs, out_shape):
    return pl.pallas_call(
        body,
        grid=(NBLK,),
        in_specs=in_specs,
        out_specs=out_specs,
        out_shape=out_shape,
    )(*in_arrays)


def kernel(x, edge_index, batch, params):
    f32 = jnp.float32
    src = edge_index[0]
    dst = edge_index[1]
    pad = EPAD - src.shape[0]
    srcpad = jnp.concatenate([src, jnp.zeros((pad,), jnp.int32)])
    srcp = [(2 * srcpad + p).reshape(NC, NS, NCHUNK, CH) for p in range(2)]
    dstp = jnp.concatenate([dst, jnp.full((pad,), N, jnp.int32)]).reshape(NC, NS, NCHUNK, CH)
    batchp = jnp.concatenate([batch, jnp.full((NP - N,), G, jnp.int32)])

    deg_parts = _deg_kernel(dstp)
    d0 = deg_parts[0].reshape(NP, 1)
    d1 = deg_parts[1].reshape(NP, 1)

    p = params
    col_spec = pl.BlockSpec((BLK, 1), lambda i: (i, 0))
    row128 = _row_spec()
    w_spec = _full_spec((H, H))
    b_spec = _full_spec((1, H))
    nh_shape = jax.ShapeDtypeStruct((NP, H), f32)

    def b2d(v):
        return v.reshape(1, H)

    xp = jnp.pad(x, ((0, NP - x.shape[0]), (0, 0)))
    h, u = _tc_call(
        _in_body,
        [xp, p["W_in"], b2d(p["b_in"]), p["convs"][0]["W"], d0, d1],
        [row128, _full_spec((x.shape[1], H)), b_spec, w_spec, col_spec, col_spec],
        [row128, row128],
        [nh_shape, nh_shape],
    )

    row64 = _row_spec(HB)
    for i in range(L):
        u2 = u.reshape(2 * NP, HB)
        acc = _edge_kernel(u2, srcp[0], srcp[1], dstp)
        halves = [acc[0, 0], acc[0, 1], acc[1, 0], acc[1, 1]]
        conv_b = b2d(p["convs"][i]["b"])
        bn_g = b2d(p["bns"][i]["g"])
        bn_b = b2d(p["bns"][i]["b"])
        if i < L - 1:
            h, u = _tc_call(
                _fuse_body,
                halves + [u, h, d0, d1, conv_b, bn_g, bn_b, p["convs"][i + 1]["W"]],
                [row64, row64, row64, row64, row128, row128, col_spec, col_spec,
                 b_spec, b_spec, b_spec, w_spec],
                [row128, row128],
                [nh_shape, nh_shape],
            )
        else:
            h = _tc_call(
                _last_body,
                halves + [u, h, d0, d1, conv_b, bn_g, bn_b],
                [row64, row64, row64, row64, row128, row128, col_spec, col_spec,
                 b_spec, b_spec, b_spec],
                row128,
                nh_shape,
            )

    mean, mx = _pool_kernel(h, batchp)

    m = p["mlp"]
    w1a = m["W1"][:H]
    w1b = m["W1"][H:]
    w2p = jnp.pad(m["W2"], ((0, 0), (0, H // 2)))
    b2p = jnp.pad(m["b2"], (0, H // 2))
    g2p = jnp.pad(m["g2"], (0, H // 2))
    bt2p = jnp.pad(m["bt2"], (0, H // 2))
    w3p = jnp.pad(m["W3"], ((0, H // 2), (0, H - m["W3"].shape[1])))
    b3p = jnp.pad(m["b3"], (0, H - m["b3"].shape[0]))

    gs = _full_spec((G, H))
    out = pl.pallas_call(
        _mlp_body,
        grid=(1,),
        in_specs=[gs, gs, w_spec, w_spec, b_spec, b_spec, b_spec, w_spec,
                  b_spec, b_spec, b_spec, w_spec, b_spec],
        out_specs=gs,
        out_shape=jax.ShapeDtypeStruct((G, H), f32),
    )(mean, mx, w1a, w1b, b2d(m["b1"]), b2d(m["g1"]), b2d(m["bt1"]),
      w2p, b2d(b2p), b2d(g2p), b2d(bt2p), w3p, b2d(b3p))

    return out[:, :m["W3"].shape[1]]


# speedup vs baseline: 2.9642x; 2.9642x over previous
import functools
import math

import jax
import jax.numpy as jnp
from jax import lax
from jax.experimental import pallas as pl
from jax.experimental.pallas import tpu as pltpu
from jax.experimental.pallas import tpu_sc as plsc

N = 10000
H = 128
G = 64
L = 4
EPS = 1e-5
CBN = 1.0 / math.sqrt(1.0 + EPS)
NEG = -3.0e38

NC, NS = 2, 16
NW = NC * NS
NP = 10240
RPT = NP // NS
CH = 128
NCHUNK = 80
EPT = NCHUNK * CH
EPAD = NW * EPT
HB = H // 2
NBUF = 4

BLK = 512
NBLK = NP // BLK

_mesh = plsc.VectorSubcoreMesh(core_axis_name="c", subcore_axis_name="s")


def _zero_vmem_rows(ref, nrows, width):
    zz = jnp.zeros((16,), jnp.float32)

    def body(r, _):
        for k in range(width // 16):
            ref[r, pl.ds(k * 16, 16)] = zz
        return 0

    lax.fori_loop(0, nrows, body, 0)


@functools.partial(
    pl.kernel,
    out_type=jax.ShapeDtypeStruct((NC, NP), jnp.float32),
    mesh=_mesh,
    scratch_types=[
        pltpu.VMEM((NCHUNK, CH), jnp.int32),
        pltpu.VMEM((CH,), jnp.float32),
        pltpu.VMEM((RPT,), jnp.float32),
        pltpu.VMEM_SHARED((NP,), jnp.float32),
    ],
    compiler_params=pltpu.CompilerParams(use_tc_tiling_on_sc=False),
)
def _deg_kernel(dst_hbm, out_hbm, idx_v, ones_v, zrow_v, acc_sh):
    c = lax.axis_index("c")
    s = lax.axis_index("s")
    pltpu.sync_copy(dst_hbm.at[c, s], idx_v)
    one = jnp.ones((16,), jnp.float32)
    zero = jnp.zeros((16,), jnp.float32)
    for k in range(CH // 16):
        ones_v[pl.ds(k * 16, 16)] = one

    def zb(i, _):
        zrow_v[pl.ds(i * 16, 16)] = zero
        return 0

    lax.fori_loop(0, RPT // 16, zb, 0)
    pltpu.sync_copy(zrow_v, acc_sh.at[pl.ds(s * RPT, RPT)])
    plsc.subcore_barrier()

    def body(j, _):
        pltpu.sync_copy(ones_v, acc_sh.at[idx_v.at[j]], add=True)
        return 0

    lax.fori_loop(0, NCHUNK, body, 0)
    plsc.subcore_barrier()
    pltpu.sync_copy(acc_sh.at[pl.ds(s * RPT, RPT)], out_hbm.at[c, pl.ds(s * RPT, RPT)])


@functools.partial(
    pl.kernel,
    out_type=jax.ShapeDtypeStruct((2, NC, NP, HB), jnp.float32),
    mesh=_mesh,
    scratch_types=[
        pltpu.VMEM((NCHUNK, CH), jnp.int32),
        pltpu.VMEM((NCHUNK, CH), jnp.int32),
        [pltpu.VMEM((CH, HB), jnp.float32) for _ in range(NBUF)],
        pltpu.VMEM((CH, HB), jnp.float32),
        pltpu.VMEM_SHARED((NP, HB), jnp.float32),
        [pltpu.SemaphoreType.DMA for _ in range(NBUF)],
    ],
    compiler_params=pltpu.CompilerParams(use_tc_tiling_on_sc=False),
)
def _edge_kernel(u_hbm, src0_hbm, src1_hbm, dst_hbm, out_hbm, src_v, dst_v,
                 rows, zbuf_v, acc_sh, sems):
    c = lax.axis_index("c")
    s = lax.axis_index("s")
    pltpu.sync_copy(dst_hbm.at[c, s], dst_v)
    _zero_vmem_rows(zbuf_v, CH, HB)

    for p, src_hbm in enumerate((src0_hbm, src1_hbm)):
        pltpu.sync_copy(src_hbm.at[c, s], src_v)
        for j in range(RPT // CH):
            pltpu.sync_copy(zbuf_v, acc_sh.at[pl.ds(s * RPT + j * CH, CH)])
        plsc.subcore_barrier()

        for b in range(NBUF):
            pltpu.async_copy(u_hbm.at[src_v.at[b]], rows[b], sems[b])

        def body(jj, _):
            j = jj * NBUF
            for b in range(NBUF):
                pltpu.make_async_copy(u_hbm.at[src_v.at[0]], rows[b], sems[b]).wait()
                pltpu.sync_copy(rows[b], acc_sh.at[dst_v.at[j + b]], add=True)
                pltpu.async_copy(u_hbm.at[src_v.at[j + b + NBUF]], rows[b], sems[b])
            return 0

        lax.fori_loop(0, NCHUNK // NBUF - 1, body, 0)
        for b in range(NBUF):
            pltpu.make_async_copy(u_hbm.at[src_v.at[0]], rows[b], sems[b]).wait()
            pltpu.sync_copy(rows[b], acc_sh.at[dst_v.at[NCHUNK - NBUF + b]], add=True)

        plsc.subcore_barrier()
        for j in range(RPT // CH):
            r0 = s * RPT + j * CH
            pltpu.sync_copy(acc_sh.at[pl.ds(r0, CH)], out_hbm.at[p, c, pl.ds(r0, CH)])


@functools.partial(
    pl.kernel,
    out_type=(
        jax.ShapeDtypeStruct((G, H), jnp.float32),
        jax.ShapeDtypeStruct((G, H), jnp.float32),
    ),
    mesh=_mesh,
    scratch_types=[
        pltpu.VMEM((NP,), jnp.int32),
        pltpu.VMEM((16, H), jnp.float32),
        pltpu.VMEM((2, H), jnp.float32),
        pltpu.VMEM((2, H), jnp.float32),
    ],
    compiler_params=pltpu.CompilerParams(
        use_tc_tiling_on_sc=False, needs_layout_passes=False),
)
def _pool_kernel(h_hbm, batch_hbm, mean_hbm, mx_hbm, batch_v, rowbuf, mbuf, xbuf):
    c = lax.axis_index("c")
    s = lax.axis_index("s")
    wid = s * NC + c
    g0 = 2 * wid
    pltpu.sync_copy(batch_hbm, batch_v)
    i0 = jnp.int32(0)
    i1 = jnp.int32(1)

    def cnt_body(i, carry):
        lt0, n0, n1 = carry
        b = batch_v[pl.ds(i * 16, 16)]
        vg0 = jnp.full((16,), g0, jnp.int32)
        vg1 = jnp.full((16,), g0 + 1, jnp.int32)
        vz = jnp.zeros((16,), jnp.int32)
        vo = jnp.full((16,), 1, jnp.int32)
        lt = jnp.minimum(jnp.maximum(vg0 - b, vz), vo)
        e0 = vo - jnp.minimum(jnp.abs(b - vg0), vo)
        e1 = vo - jnp.minimum(jnp.abs(b - vg1), vo)
        return lt0 + jnp.sum(lt), n0 + jnp.sum(e0), n1 + jnp.sum(e1)

    lt0, n0, n1 = lax.fori_loop(0, NP // 16, cnt_body, (i0, i0, i0))

    def seg_reduce(start, n, row):
        zero = jnp.zeros((16,), jnp.float32)
        negv = jnp.full((16,), NEG, jnp.float32)
        for k in range(H // 16):
            mbuf[row, pl.ds(k * 16, 16)] = zero
            xbuf[row, pl.ds(k * 16, 16)] = negv

        def ch_body(j, _):
            pltpu.sync_copy(h_hbm.at[pl.ds(start + j * 16, 16)], rowbuf)
            vz = jnp.zeros((16,), jnp.int32)
            vo = jnp.full((16,), 1, jnp.int32)
            vn = jnp.full((16,), n, jnp.int32)
            for r in range(16):
                vr = jnp.full((16,), j * 16 + r, jnp.int32)
                vf = jnp.minimum(jnp.maximum(vn - vr, vz), vo).astype(jnp.float32)
                off = (1.0 - vf) * NEG
                for k in range(H // 16):
                    v = rowbuf[r, pl.ds(k * 16, 16)]
                    mbuf[row, pl.ds(k * 16, 16)] = mbuf[row, pl.ds(k * 16, 16)] + v * vf
                    xbuf[row, pl.ds(k * 16, 16)] = jnp.maximum(
                        xbuf[row, pl.ds(k * 16, 16)], v * vf + off)
            return 0

        nch = lax.shift_right_logical(n + 15, 4)
        lax.fori_loop(0, nch, ch_body, 0)
        vn = jnp.full((16,), n, jnp.int32)
        nzv = jnp.minimum(vn, jnp.full((16,), 1, jnp.int32)).astype(jnp.float32)
        inv = nzv / jnp.maximum(vn.astype(jnp.float32), jnp.full((16,), 1.0))
        for k in range(H // 16):
            mbuf[row, pl.ds(k * 16, 16)] = mbuf[row, pl.ds(k * 16, 16)] * inv
            xbuf[row, pl.ds(k * 16, 16)] = xbuf[row, pl.ds(k * 16, 16)] * nzv

    seg_reduce(lt0, n0, 0)
    seg_reduce(lt0 + n0, n1, 1)
    pltpu.sync_copy(mbuf, mean_hbm.at[pl.ds(g0, 2)])
    pltpu.sync_copy(xbuf, mx_hbm.at[pl.ds(g0, 2)])


def _row_spec(width=H):
    return pl.BlockSpec((BLK, width), lambda i: (i, 0))


def _full_spec(shape):
    nd = len(shape)
    return pl.BlockSpec(shape, lambda i: (0,) * nd)


def _dinv(d0, d1):
    return lax.rsqrt(d0 + d1 + 1.0)


def _in_body(x_ref, win_ref, bin_ref, w1_ref, d0_ref, d1_ref, h_ref, u_ref):
    h = jnp.maximum(jnp.dot(x_ref[...], win_ref[...],
                            preferred_element_type=jnp.float32) + bin_ref[...], 0.0)
    dinv = _dinv(d0_ref[...], d1_ref[...])
    h_ref[...] = h
    u_ref[...] = dinv * jnp.dot(h, w1_ref[...], preferred_element_type=jnp.float32)


def _acc_full(a00_ref, a01_ref, a10_ref, a11_ref):
    return jnp.concatenate(
        [a00_ref[...] + a01_ref[...], a10_ref[...] + a11_ref[...]], axis=1)


def _fuse_body(a00_ref, a01_ref, a10_ref, a11_ref, u_ref, res_ref, d0_ref,
               d1_ref, b_ref, g_ref, bt_ref, wn_ref, h_ref, un_ref):
    dinv = _dinv(d0_ref[...], d1_ref[...])
    acc = _acc_full(a00_ref, a01_ref, a10_ref, a11_ref)
    t = dinv * (acc + u_ref[...]) + b_ref[...]
    t = t * (CBN * g_ref[...]) + bt_ref[...]
    h = jnp.maximum(t, 0.0) + res_ref[...]
    h_ref[...] = h
    un_ref[...] = dinv * jnp.dot(h, wn_ref[...], preferred_element_type=jnp.float32)


def _last_body(a00_ref, a01_ref, a10_ref, a11_ref, u_ref, res_ref, d0_ref,
               d1_ref, b_ref, g_ref, bt_ref, h_ref):
    dinv = _dinv(d0_ref[...], d1_ref[...])
    acc = _acc_full(a00_ref, a01_ref, a10_ref, a11_ref)
    t = dinv * (acc + u_ref[...]) + b_ref[...]
    t = t * (CBN * g_ref[...]) + bt_ref[...]
    h_ref[...] = jnp.maximum(t, 0.0) + res_ref[...]


def _mlp_body(mean_ref, mx_ref, w1a_ref, w1b_ref, b1_ref, g1_ref, bt1_ref,
              w2_ref, b2_ref, g2_ref, bt2_ref, w3_ref, b3_ref, out_ref):
    z = (jnp.dot(mean_ref[...], w1a_ref[...], preferred_element_type=jnp.float32)
         + jnp.dot(mx_ref[...], w1b_ref[...], preferred_element_type=jnp.float32)
         + b1_ref[...])
    z = jnp.maximum(z * (CBN * g1_ref[...]) + bt1_ref[...], 0.0)
    z = jnp.dot(z, w2_ref[...], preferred_element_type=jnp.float32) + b2_ref[...]
    z = jnp.maximum(z * (CBN * g2_ref[...]) + bt2_ref[...], 0.0)
    out_ref[...] = jnp.dot(z, w3_ref[...], preferred_element_type=jnp.float32) + b3_ref[...]


def _tc_call(body, in_arrays, in_specs, out_specs, out_shape):
    return pl.pallas_call(
        body,
        grid=(NBLK,),
        in_specs=in_specs,
        out_specs=out_specs,
        out_shape=out_shape,
    )(*in_arrays)


def kernel(x, edge_index, batch, params):
    f32 = jnp.float32
    src = edge_index[0]
    dst = edge_index[1]
    pad = EPAD - src.shape[0]
    padix = jax.lax.iota(jnp.int32, pad)
    srcpad = jnp.concatenate([src, padix % N])
    srcp = [(2 * srcpad + p).reshape(NC, NS, NCHUNK, CH) for p in range(2)]
    dstp = jnp.concatenate([dst, N + padix % (NP - N)]).reshape(NC, NS, NCHUNK, CH)
    batchp = jnp.concatenate([batch, jnp.full((NP - N,), G, jnp.int32)])

    deg_parts = _deg_kernel(dstp)
    d0 = deg_parts[0].reshape(NP, 1)
    d1 = deg_parts[1].reshape(NP, 1)

    p = params
    col_spec = pl.BlockSpec((BLK, 1), lambda i: (i, 0))
    row128 = _row_spec()
    w_spec = _full_spec((H, H))
    b_spec = _full_spec((1, H))
    nh_shape = jax.ShapeDtypeStruct((NP, H), f32)

    def b2d(v):
        return v.reshape(1, H)

    xp = jnp.pad(x, ((0, NP - x.shape[0]), (0, 0)))
    h, u = _tc_call(
        _in_body,
        [xp, p["W_in"], b2d(p["b_in"]), p["convs"][0]["W"], d0, d1],
        [row128, _full_spec((x.shape[1], H)), b_spec, w_spec, col_spec, col_spec],
        [row128, row128],
        [nh_shape, nh_shape],
    )

    row64 = _row_spec(HB)
    for i in range(L):
        u2 = u.reshape(2 * NP, HB)
        acc = _edge_kernel(u2, srcp[0], srcp[1], dstp)
        halves = [acc[0, 0], acc[0, 1], acc[1, 0], acc[1, 1]]
        conv_b = b2d(p["convs"][i]["b"])
        bn_g = b2d(p["bns"][i]["g"])
        bn_b = b2d(p["bns"][i]["b"])
        if i < L - 1:
            h, u = _tc_call(
                _fuse_body,
                halves + [u, h, d0, d1, conv_b, bn_g, bn_b, p["convs"][i + 1]["W"]],
                [row64, row64, row64, row64, row128, row128, col_spec, col_spec,
                 b_spec, b_spec, b_spec, w_spec],
                [row128, row128],
                [nh_shape, nh_shape],
            )
        else:
            h = _tc_call(
                _last_body,
                halves + [u, h, d0, d1, conv_b, bn_g, bn_b],
                [row64, row64, row64, row64, row128, row128, col_spec, col_spec,
                 b_spec, b_spec, b_spec],
                row128,
                nh_shape,
            )

    mean, mx = _pool_kernel(h, batchp)

    m = p["mlp"]
    w1a = m["W1"][:H]
    w1b = m["W1"][H:]
    w2p = jnp.pad(m["W2"], ((0, 0), (0, H // 2)))
    b2p = jnp.pad(m["b2"], (0, H // 2))
    g2p = jnp.pad(m["g2"], (0, H // 2))
    bt2p = jnp.pad(m["bt2"], (0, H // 2))
    w3p = jnp.pad(m["W3"], ((0, H // 2), (0, H - m["W3"].shape[1])))
    b3p = jnp.pad(m["b3"], (0, H - m["b3"].shape[0]))

    gs = _full_spec((G, H))
    out = pl.pallas_call(
        _mlp_body,
        grid=(1,),
        in_specs=[gs, gs, w_spec, w_spec, b_spec, b_spec, b_spec, w_spec,
                  b_spec, b_spec, b_spec, w_spec, b_spec],
        out_specs=gs,
        out_shape=jax.ShapeDtypeStruct((G, H), f32),
    )(mean, mx, w1a, w1b, b2d(m["b1"]), b2d(m["g1"]), b2d(m["bt1"]),
      w2p, b2d(b2p), b2d(g2p), b2d(bt2p), w3p, b2d(b3p))

    return out[:, :m["W3"].shape[1]]


# speedup vs baseline: 3.6670x; 1.2371x over previous
import functools
import math

import jax
import jax.numpy as jnp
from jax import lax
from jax.experimental import pallas as pl
from jax.experimental.pallas import tpu as pltpu
from jax.experimental.pallas import tpu_sc as plsc

N = 10000
H = 128
G = 64
L = 4
EPS = 1e-5
CBN = 1.0 / math.sqrt(1.0 + EPS)
NEG = -3.0e38

NC, NS = 2, 16
NW = NC * NS
NP = 10240
RPT = NP // NS
CH = 128
NCHUNK = 80
EPT = NCHUNK * CH
EPAD = NW * EPT
HB = H // 2
NBUF = 4
HALF = NP // 2
DUMR = 256
ACCR = HALF + DUMR
CAPC = NCHUNK + 1
CAP = CAPC * CH
RZT = ACCR // NS
RCT = HALF // NS

BLK = 512
NBLK = NP // BLK

_mesh = plsc.VectorSubcoreMesh(core_axis_name="c", subcore_axis_name="s")


def _zero_vmem_rows(ref, nrows, width):
    zz = jnp.zeros((16,), jnp.float32)

    def body(r, _):
        for k in range(width // 16):
            ref[r, pl.ds(k * 16, 16)] = zz
        return 0

    lax.fori_loop(0, nrows, body, 0)


@functools.partial(
    pl.kernel,
    out_type=jax.ShapeDtypeStruct((NC, NP), jnp.float32),
    mesh=_mesh,
    scratch_types=[
        pltpu.VMEM((NCHUNK, CH), jnp.int32),
        pltpu.VMEM((CH,), jnp.float32),
        pltpu.VMEM((RPT,), jnp.float32),
        pltpu.VMEM_SHARED((NP,), jnp.float32),
    ],
    compiler_params=pltpu.CompilerParams(use_tc_tiling_on_sc=False),
)
def _deg_kernel(dst_hbm, out_hbm, idx_v, ones_v, zrow_v, acc_sh):
    c = lax.axis_index("c")
    s = lax.axis_index("s")
    pltpu.sync_copy(dst_hbm.at[c, s], idx_v)
    one = jnp.ones((16,), jnp.float32)
    zero = jnp.zeros((16,), jnp.float32)
    for k in range(CH // 16):
        ones_v[pl.ds(k * 16, 16)] = one

    def zb(i, _):
        zrow_v[pl.ds(i * 16, 16)] = zero
        return 0

    lax.fori_loop(0, RPT // 16, zb, 0)
    pltpu.sync_copy(zrow_v, acc_sh.at[pl.ds(s * RPT, RPT)])
    plsc.subcore_barrier()

    def body(j, _):
        pltpu.sync_copy(ones_v, acc_sh.at[idx_v.at[j]], add=True)
        return 0

    lax.fori_loop(0, NCHUNK, body, 0)
    plsc.subcore_barrier()
    pltpu.sync_copy(acc_sh.at[pl.ds(s * RPT, RPT)], out_hbm.at[c, pl.ds(s * RPT, RPT)])


@functools.partial(
    pl.kernel,
    out_type=(
        jax.ShapeDtypeStruct((2, NW, CAPC, CH), jnp.int32),
        jax.ShapeDtypeStruct((2, NW, CAPC, CH), jnp.int32),
        jax.ShapeDtypeStruct((2, NW, 16), jnp.int32),
    ),
    mesh=_mesh,
    scratch_types=[
        pltpu.VMEM((EPT,), jnp.int32),
        pltpu.VMEM((EPT,), jnp.int32),
        pltpu.VMEM((CAPC, CH), jnp.int32),
        pltpu.VMEM((CAPC, CH), jnp.int32),
        pltpu.VMEM((CAPC, CH), jnp.int32),
        pltpu.VMEM((CAPC, CH), jnp.int32),
        pltpu.VMEM((16,), jnp.int32),
    ],
    compiler_params=pltpu.CompilerParams(
        use_tc_tiling_on_sc=False, needs_layout_passes=False),
)
def _prep_kernel(srcE, dstE, srcL, dstL, cnts, sv, dv, sl0, dl0, sl1, dl1, cb):
    c = lax.axis_index("c")
    s = lax.axis_index("s")
    t = s * NC + c
    pltpu.sync_copy(srcE.at[t], sv)
    pltpu.sync_copy(dstE.at[t], dv)
    i0 = jnp.int32(0)
    m127 = jnp.full((16,), 127, jnp.int32)
    trash = jnp.full((16,), CAP - 1, jnp.int32)

    def part(i, carry):
        off0, off1 = carry
        svv = sv[pl.ds(i * 16, 16)]
        dvv = dv[pl.ds(i * 16, 16)]
        vz = jnp.zeros((16,), jnp.int32)
        vo = jnp.full((16,), 1, jnp.int32)
        m1 = jnp.minimum(jnp.maximum(dvv - (HALF - 1), vz), vo)
        m0 = vo - m1
        pos0 = jnp.full((16,), off0, jnp.int32) + plsc.cumsum(m0) - m0
        pos1 = jnp.full((16,), off1, jnp.int32) + plsc.cumsum(m1) - m1
        t0 = m0 * pos0 + m1 * trash
        t1 = m1 * pos1 + m0 * trash
        r0 = lax.shift_right_logical(t0, 7)
        q0 = jnp.bitwise_and(t0, m127)
        r1 = lax.shift_right_logical(t1, 7)
        q1 = jnp.bitwise_and(t1, m127)
        plsc.store_scatter(sl0, [r0, q0], svv)
        plsc.store_scatter(dl0, [r0, q0], dvv)
        plsc.store_scatter(sl1, [r1, q1], svv)
        plsc.store_scatter(dl1, [r1, q1], dvv - jnp.full((16,), HALF, jnp.int32))
        return off0 + jnp.sum(m0), off1 + jnp.sum(m1)

    off0, off1 = lax.fori_loop(0, EPT // 16, part, (i0, i0))

    def pad_list(sl, dl, off):
        padded = lax.shift_left(lax.shift_right_logical(off + 127, 7), 7)
        n16 = lax.shift_right_logical(padded - off + 15, 4)

        def pb(j, _):
            lanes = lax.iota(jnp.int32, 16) + jnp.full((16,), off + j * 16, jnp.int32)
            over = jnp.minimum(jnp.maximum(
                lanes - jnp.full((16,), padded - 1, jnp.int32),
                jnp.zeros((16,), jnp.int32)), jnp.full((16,), 1, jnp.int32))
            pos = (jnp.full((16,), 1, jnp.int32) - over) * lanes + over * trash
            r = lax.shift_right_logical(pos, 7)
            q = jnp.bitwise_and(pos, m127)
            dum_d = jnp.full((16,), HALF, jnp.int32) + jnp.bitwise_and(
                lanes, jnp.full((16,), DUMR - 1, jnp.int32))
            dum_s = jnp.bitwise_and(lanes, jnp.full((16,), 8191, jnp.int32))
            plsc.store_scatter(sl, [r, q], dum_s)
            plsc.store_scatter(dl, [r, q], dum_d)
            return 0

        lax.fori_loop(0, n16, pb, 0)
        return lax.shift_right_logical(padded, 7)

    nch0 = pad_list(sl0, dl0, off0)
    nch1 = pad_list(sl1, dl1, off1)
    cb[pl.ds(0, 16)] = jnp.full((16,), nch0, jnp.int32)
    pltpu.sync_copy(cb, cnts.at[0, t])
    pltpu.sync_copy(sl0, srcL.at[0, t])
    pltpu.sync_copy(dl0, dstL.at[0, t])
    cb[pl.ds(0, 16)] = jnp.full((16,), nch1, jnp.int32)
    pltpu.sync_copy(cb, cnts.at[1, t])
    pltpu.sync_copy(sl1, srcL.at[1, t])
    pltpu.sync_copy(dl1, dstL.at[1, t])


@functools.partial(
    pl.kernel,
    out_type=jax.ShapeDtypeStruct((NP, H), jnp.float32),
    mesh=_mesh,
    scratch_types=[
        pltpu.VMEM((CAPC, CH), jnp.int32),
        pltpu.VMEM((CAPC, CH), jnp.int32),
        [pltpu.VMEM((CH, H), jnp.float32) for _ in range(NBUF)],
        pltpu.VMEM((16,), jnp.int32),
        pltpu.VMEM_SHARED((ACCR, H), jnp.float32),
        [pltpu.SemaphoreType.DMA for _ in range(NBUF)],
    ],
    compiler_params=pltpu.CompilerParams(
        use_tc_tiling_on_sc=False, needs_layout_passes=False),
)
def _edge_kernel(u_hbm, srcL, dstL, cnts, out_hbm, sv2, dv2, rows,
                 csm, acc_sh, sems):
    c = lax.axis_index("c")
    s = lax.axis_index("s")
    _zero_vmem_rows(rows[0], CH, H)
    base = s * RZT
    pltpu.sync_copy(rows[0], acc_sh.at[pl.ds(base, CH)])
    pltpu.sync_copy(rows[0], acc_sh.at[pl.ds(base + CH, CH)])
    pltpu.sync_copy(rows[0].at[pl.ds(0, RZT - 2 * CH)],
                    acc_sh.at[pl.ds(base + 2 * CH, RZT - 2 * CH)])
    plsc.subcore_barrier()

    for reg in (2 * s, 2 * s + 1):
        pltpu.sync_copy(cnts.at[c, reg], csm)
        trip = jnp.max(csm[pl.ds(0, 16)])
        pltpu.sync_copy(srcL.at[c, reg], sv2)
        pltpu.sync_copy(dstL.at[c, reg], dv2)
        for b in range(NBUF):
            @pl.when(b < trip)
            def _():
                pltpu.async_copy(u_hbm.at[sv2.at[b]], rows[b], sems[b])

        def rbody(jj, _):
            j0 = jj * NBUF
            for b in range(NBUF):
                idx = j0 + b

                @pl.when(idx < trip)
                def _():
                    pltpu.make_async_copy(u_hbm.at[sv2.at[0]], rows[b], sems[b]).wait()
                    pltpu.sync_copy(rows[b], acc_sh.at[dv2.at[idx]], add=True)

                    @pl.when(idx + NBUF < trip)
                    def _():
                        pltpu.async_copy(u_hbm.at[sv2.at[idx + NBUF]], rows[b], sems[b])
            return 0

        ntrips = lax.shift_right_logical(trip + (NBUF - 1), 2)
        lax.fori_loop(0, ntrips, rbody, 0)

    plsc.subcore_barrier()
    lb = s * RCT
    g0 = c * HALF + lb
    pltpu.sync_copy(acc_sh.at[pl.ds(lb, CH)], out_hbm.at[pl.ds(g0, CH)])
    pltpu.sync_copy(acc_sh.at[pl.ds(lb + CH, CH)], out_hbm.at[pl.ds(g0 + CH, CH)])
    pltpu.sync_copy(acc_sh.at[pl.ds(lb + 2 * CH, RCT - 2 * CH)],
                    out_hbm.at[pl.ds(g0 + 2 * CH, RCT - 2 * CH)])


@functools.partial(
    pl.kernel,
    out_type=(
        jax.ShapeDtypeStruct((G, H), jnp.float32),
        jax.ShapeDtypeStruct((G, H), jnp.float32),
    ),
    mesh=_mesh,
    scratch_types=[
        pltpu.VMEM((NP,), jnp.int32),
        pltpu.VMEM((16, H), jnp.float32),
        pltpu.VMEM((2, H), jnp.float32),
        pltpu.VMEM((2, H), jnp.float32),
    ],
    compiler_params=pltpu.CompilerParams(
        use_tc_tiling_on_sc=False, needs_layout_passes=False),
)
def _pool_kernel(h_hbm, batch_hbm, mean_hbm, mx_hbm, batch_v, rowbuf, mbuf, xbuf):
    c = lax.axis_index("c")
    s = lax.axis_index("s")
    wid = s * NC + c
    g0 = 2 * wid
    pltpu.sync_copy(batch_hbm, batch_v)
    i0 = jnp.int32(0)
    i1 = jnp.int32(1)

    def cnt_body(i, carry):
        lt0, n0, n1 = carry
        b = batch_v[pl.ds(i * 16, 16)]
        vg0 = jnp.full((16,), g0, jnp.int32)
        vg1 = jnp.full((16,), g0 + 1, jnp.int32)
        vz = jnp.zeros((16,), jnp.int32)
        vo = jnp.full((16,), 1, jnp.int32)
        lt = jnp.minimum(jnp.maximum(vg0 - b, vz), vo)
        e0 = vo - jnp.minimum(jnp.abs(b - vg0), vo)
        e1 = vo - jnp.minimum(jnp.abs(b - vg1), vo)
        return lt0 + jnp.sum(lt), n0 + jnp.sum(e0), n1 + jnp.sum(e1)

    lt0, n0, n1 = lax.fori_loop(0, NP // 16, cnt_body, (i0, i0, i0))

    def seg_reduce(start, n, row):
        zero = jnp.zeros((16,), jnp.float32)
        negv = jnp.full((16,), NEG, jnp.float32)
        for k in range(H // 16):
            mbuf[row, pl.ds(k * 16, 16)] = zero
            xbuf[row, pl.ds(k * 16, 16)] = negv

        def ch_body(j, _):
            pltpu.sync_copy(h_hbm.at[pl.ds(start + j * 16, 16)], rowbuf)
            vz = jnp.zeros((16,), jnp.int32)
            vo = jnp.full((16,), 1, jnp.int32)
            vn = jnp.full((16,), n, jnp.int32)
            for r in range(16):
                vr = jnp.full((16,), j * 16 + r, jnp.int32)
                vf = jnp.minimum(jnp.maximum(vn - vr, vz), vo).astype(jnp.float32)
                off = (1.0 - vf) * NEG
                for k in range(H // 16):
                    v = rowbuf[r, pl.ds(k * 16, 16)]
                    mbuf[row, pl.ds(k * 16, 16)] = mbuf[row, pl.ds(k * 16, 16)] + v * vf
                    xbuf[row, pl.ds(k * 16, 16)] = jnp.maximum(
                        xbuf[row, pl.ds(k * 16, 16)], v * vf + off)
            return 0

        nch = lax.shift_right_logical(n + 15, 4)
        lax.fori_loop(0, nch, ch_body, 0)
        vn = jnp.full((16,), n, jnp.int32)
        nzv = jnp.minimum(vn, jnp.full((16,), 1, jnp.int32)).astype(jnp.float32)
        inv = nzv / jnp.maximum(vn.astype(jnp.float32), jnp.full((16,), 1.0))
        for k in range(H // 16):
            mbuf[row, pl.ds(k * 16, 16)] = mbuf[row, pl.ds(k * 16, 16)] * inv
            xbuf[row, pl.ds(k * 16, 16)] = xbuf[row, pl.ds(k * 16, 16)] * nzv

    seg_reduce(lt0, n0, 0)
    seg_reduce(lt0 + n0, n1, 1)
    pltpu.sync_copy(mbuf, mean_hbm.at[pl.ds(g0, 2)])
    pltpu.sync_copy(xbuf, mx_hbm.at[pl.ds(g0, 2)])


def _row_spec(width=H):
    return pl.BlockSpec((BLK, width), lambda i: (i, 0))


def _full_spec(shape):
    nd = len(shape)
    return pl.BlockSpec(shape, lambda i: (0,) * nd)


def _dinv(d0, d1):
    return lax.rsqrt(d0 + d1 + 1.0)


def _in_body(x_ref, win_ref, bin_ref, w1_ref, d0_ref, d1_ref, h_ref, u_ref):
    h = jnp.maximum(jnp.dot(x_ref[...], win_ref[...],
                            preferred_element_type=jnp.float32) + bin_ref[...], 0.0)
    dinv = _dinv(d0_ref[...], d1_ref[...])
    h_ref[...] = h
    u_ref[...] = dinv * jnp.dot(h, w1_ref[...], preferred_element_type=jnp.float32)


def _fuse_body(acc_ref, u_ref, res_ref, d0_ref, d1_ref, b_ref, g_ref, bt_ref,
               wn_ref, h_ref, un_ref):
    dinv = _dinv(d0_ref[...], d1_ref[...])
    t = dinv * (acc_ref[...] + u_ref[...]) + b_ref[...]
    t = t * (CBN * g_ref[...]) + bt_ref[...]
    h = jnp.maximum(t, 0.0) + res_ref[...]
    h_ref[...] = h
    un_ref[...] = dinv * jnp.dot(h, wn_ref[...], preferred_element_type=jnp.float32)


def _last_body(acc_ref, u_ref, res_ref, d0_ref, d1_ref, b_ref, g_ref, bt_ref,
               h_ref):
    dinv = _dinv(d0_ref[...], d1_ref[...])
    t = dinv * (acc_ref[...] + u_ref[...]) + b_ref[...]
    t = t * (CBN * g_ref[...]) + bt_ref[...]
    h_ref[...] = jnp.maximum(t, 0.0) + res_ref[...]


def _mlp_body(mean_ref, mx_ref, w1a_ref, w1b_ref, b1_ref, g1_ref, bt1_ref,
              w2_ref, b2_ref, g2_ref, bt2_ref, w3_ref, b3_ref, out_ref):
    z = (jnp.dot(mean_ref[...], w1a_ref[...], preferred_element_type=jnp.float32)
         + jnp.dot(mx_ref[...], w1b_ref[...], preferred_element_type=jnp.float32)
         + b1_ref[...])
    z = jnp.maximum(z * (CBN * g1_ref[...]) + bt1_ref[...], 0.0)
    z = jnp.dot(z, w2_ref[...], preferred_element_type=jnp.float32) + b2_ref[...]
    z = jnp.maximum(z * (CBN * g2_ref[...]) + bt2_ref[...], 0.0)
    out_ref[...] = jnp.dot(z, w3_ref[...], preferred_element_type=jnp.float32) + b3_ref[...]


def _tc_call(body, in_arrays, in_specs, out_specs, out_shape):
    return pl.pallas_call(
        body,
        grid=(NBLK,),
        in_specs=in_specs,
        out_specs=out_specs,
        out_shape=out_shape,
    )(*in_arrays)


def kernel(x, edge_index, batch, params):
    f32 = jnp.float32
    src = edge_index[0]
    dst = edge_index[1]
    pad = EPAD - src.shape[0]
    padix = jax.lax.iota(jnp.int32, pad)
    ppb = pad // NW
    srcE = jnp.concatenate(
        [src.reshape(NW, src.shape[0] // NW), (padix % N).reshape(NW, ppb)],
        axis=1)
    dstE = jnp.concatenate(
        [dst.reshape(NW, dst.shape[0] // NW),
         (N + padix % (NP - N)).reshape(NW, ppb)], axis=1)
    dstp = dstE.reshape(NC, NS, NCHUNK, CH)
    batchp = jnp.concatenate([batch, jnp.full((NP - N,), G, jnp.int32)])

    deg_parts = _deg_kernel(dstp)
    d0 = deg_parts[0].reshape(NP, 1)
    d1 = deg_parts[1].reshape(NP, 1)

    p = params
    col_spec = pl.BlockSpec((BLK, 1), lambda i: (i, 0))
    row128 = _row_spec()
    w_spec = _full_spec((H, H))
    b_spec = _full_spec((1, H))
    nh_shape = jax.ShapeDtypeStruct((NP, H), f32)

    def b2d(v):
        return v.reshape(1, H)

    xp = jnp.pad(x, ((0, NP - x.shape[0]), (0, 0)))
    h, u = _tc_call(
        _in_body,
        [xp, p["W_in"], b2d(p["b_in"]), p["convs"][0]["W"], d0, d1],
        [row128, _full_spec((x.shape[1], H)), b_spec, w_spec, col_spec, col_spec],
        [row128, row128],
        [nh_shape, nh_shape],
    )

    srcL, dstL, cnts = _prep_kernel(srcE, dstE)
    for i in range(L):
        acc = _edge_kernel(u, srcL, dstL, cnts)
        conv_b = b2d(p["convs"][i]["b"])
        bn_g = b2d(p["bns"][i]["g"])
        bn_b = b2d(p["bns"][i]["b"])
        if i < L - 1:
            h, u = _tc_call(
                _fuse_body,
                [acc, u, h, d0, d1, conv_b, bn_g, bn_b, p["convs"][i + 1]["W"]],
                [row128, row128, row128, col_spec, col_spec,
                 b_spec, b_spec, b_spec, w_spec],
                [row128, row128],
                [nh_shape, nh_shape],
            )
        else:
            h = _tc_call(
                _last_body,
                [acc, u, h, d0, d1, conv_b, bn_g, bn_b],
                [row128, row128, row128, col_spec, col_spec,
                 b_spec, b_spec, b_spec],
                row128,
                nh_shape,
            )

    mean, mx = _pool_kernel(h, batchp)

    m = p["mlp"]
    w1a = m["W1"][:H]
    w1b = m["W1"][H:]
    w2p = jnp.pad(m["W2"], ((0, 0), (0, H // 2)))
    b2p = jnp.pad(m["b2"], (0, H // 2))
    g2p = jnp.pad(m["g2"], (0, H // 2))
    bt2p = jnp.pad(m["bt2"], (0, H // 2))
    w3p = jnp.pad(m["W3"], ((0, H // 2), (0, H - m["W3"].shape[1])))
    b3p = jnp.pad(m["b3"], (0, H - m["b3"].shape[0]))

    gs = _full_spec((G, H))
    out = pl.pallas_call(
        _mlp_body,
        grid=(1,),
        in_specs=[gs, gs, w_spec, w_spec, b_spec, b_spec, b_spec, w_spec,
                  b_spec, b_spec, b_spec, w_spec, b_spec],
        out_specs=gs,
        out_shape=jax.ShapeDtypeStruct((G, H), f32),
    )(mean, mx, w1a, w1b, b2d(m["b1"]), b2d(m["g1"]), b2d(m["bt1"]),
      w2p, b2d(b2p), b2d(g2p), b2d(bt2p), w3p, b2d(b3p))

    return out[:, :m["W3"].shape[1]]
